# Initial kernel scaffold; baseline (speedup 1.0000x reference)
import jax
import jax.numpy as jnp
from jax.experimental import pallas as pl


def kernel(loc_preds, loc_targets, cls_preds, cls_targets):
    raise NotImplementedError("write your pallas kernel here")


# speedup vs baseline: 25.5006x; 25.5006x over previous
import jax
import jax.numpy as jnp
from jax import lax
from jax.experimental import pallas as pl
from jax.experimental.pallas import tpu as pltpu

_NEG_POS_RATIO = 3
_SENTINEL = 0x7F800000


def _ssd_body(lpt_ref, ltt_ref, cp_ref, ct_ref, out_ref):
    B, N = ct_ref.shape
    ct = ct_ref[...]
    pos = ct > 0.5
    posf = pos.astype(jnp.float32)
    num_pos_i = jnp.sum(pos.astype(jnp.int32))

    loc_sum = jnp.float32(0.0)
    for c in range(4):
        d = lpt_ref[c] - ltt_ref[c]
        ad = jnp.abs(d)
        sl = jnp.where(ad < 1.0, 0.5 * d * d, ad - 0.5)
        loc_sum = loc_sum + jnp.sum(posf * sl)

    x = cp_ref[...]
    cls_elem = jnp.maximum(x, 0.0) - x * ct + jnp.log1p(jnp.exp(-jnp.abs(x)))
    cls_all_sum = jnp.sum(cls_elem)
    cls_pos_sum = jnp.sum(posf * cls_elem)

    keys = lax.bitcast_convert_type(cls_elem, jnp.int32)
    keys = jnp.where(pos, jnp.int32(_SENTINEL), keys)

    k = jnp.maximum(jnp.minimum(_NEG_POS_RATIO * num_pos_i, N - num_pos_i), 0)

    prefix0 = jnp.zeros((B, 1), jnp.int32)
    kk0 = jnp.full((B, 1), k, jnp.int32)

    def bit_step(i, carry):
        prefix, kk = carry
        bit = 30 - i
        m = (keys >> bit) == (prefix >> bit)
        c0 = jnp.sum(m.astype(jnp.int32), axis=1, keepdims=True)
        take0 = kk <= c0
        prefix = jnp.where(take0, prefix, prefix | (jnp.int32(1) << bit))
        kk = jnp.where(take0, kk, kk - c0)
        return prefix, kk

    thresh, _ = lax.fori_loop(0, 31, bit_step, (prefix0, kk0))

    vals = lax.bitcast_convert_type(keys, jnp.float32)
    v_t = lax.bitcast_convert_type(thresh, jnp.float32)
    less = keys < thresh
    cnt_less = jnp.sum(less.astype(jnp.int32), axis=1, keepdims=True)
    sum_less = jnp.sum(jnp.where(less, vals, 0.0), axis=1, keepdims=True)
    kf = k.astype(jnp.float32)
    sel = sum_less + (kf - cnt_less.astype(jnp.float32)) * v_t
    select_total = jnp.sum(sel)

    num_pos_f = num_pos_i.astype(jnp.float32)
    num_pos_safe = jnp.maximum(num_pos_f, 1.0)
    total = (loc_sum + cls_pos_sum + select_total) / num_pos_safe
    zero_branch = cls_all_sum / jnp.float32(B * N)
    result = jnp.where(num_pos_i == 0, zero_branch, total)
    out_ref[...] = jnp.broadcast_to(result, (1, 1))


def kernel(loc_preds, loc_targets, cls_preds, cls_targets):
    lpt = jnp.transpose(loc_preds, (2, 0, 1))
    ltt = jnp.transpose(loc_targets, (2, 0, 1))
    out = pl.pallas_call(
        _ssd_body,
        out_shape=jax.ShapeDtypeStruct((1, 1), jnp.float32),
    )(lpt, ltt, cls_preds, cls_targets)
    return out[0, 0]
